# Initial kernel scaffold; baseline (speedup 1.0000x reference)
import jax
import jax.numpy as jnp
from jax.experimental import pallas as pl


def kernel(q, k, v, col_descend):
    raise NotImplementedError("write your pallas kernel here")


# speedup vs baseline: 3.0873x; 3.0873x over previous
import jax
import jax.numpy as jnp
from jax.experimental import pallas as pl
from jax.experimental.pallas import tpu as pltpu


def _bitonic_sort_body(sgn_ref, v_ref, o_ref):
    S = v_ref.shape[1]
    o_ref[0] = v_ref[0] * sgn_ref[0:1, :]
    iota = jax.lax.broadcasted_iota(jnp.int32, (S, 1), 0)
    K = 2
    while K <= S:
        j = K // 2
        while j >= 1:
            x = o_ref[0]
            up = jnp.concatenate([x[j:], x[:j]], axis=0)
            down = jnp.concatenate([x[S - j:], x[:S - j]], axis=0)
            is_lo = (iota & j) == 0
            partner = jnp.where(is_lo, up, down)
            dir_asc = (iota & K) == 0
            want_min = is_lo == dir_asc
            o_ref[0] = jnp.where(want_min, jnp.minimum(x, partner),
                                 jnp.maximum(x, partner))
            j //= 2
        K *= 2
    o_ref[0] = o_ref[0] * sgn_ref[0:1, :]


def _sort_pallas(vr, sgn, *, interpret=False):
    N, S, C = vr.shape
    return pl.pallas_call(
        _bitonic_sort_body,
        grid=(N,),
        in_specs=[
            pl.BlockSpec((8, C), lambda i: (0, 0)),
            pl.BlockSpec((1, S, C), lambda i: (i, 0, 0)),
        ],
        out_specs=pl.BlockSpec((1, S, C), lambda i: (i, 0, 0)),
        out_shape=jax.ShapeDtypeStruct((N, S, C), vr.dtype),
        compiler_params=pltpu.CompilerParams(
            vmem_limit_bytes=100 * 1024 * 1024),
        interpret=interpret,
    )(sgn, vr)


def kernel(q, k, v, col_descend):
    B, H, S, C = v.shape
    mask = jnp.zeros((C,), jnp.bool_).at[col_descend.reshape(-1)].set(True)
    sgn = jnp.where(mask, -1.0, 1.0).astype(v.dtype)
    sgn = jnp.broadcast_to(sgn.reshape(1, C), (8, C))
    vr = v.reshape(B * H, S, C)
    out = _sort_pallas(vr, sgn)
    return out.reshape(B, H, S, C)
